# Initial kernel scaffold; baseline (speedup 1.0000x reference)
import jax
import jax.numpy as jnp
from jax.experimental import pallas as pl


def kernel(alpha, cu_seqlens):
    raise NotImplementedError("write your pallas kernel here")


# speedup vs baseline: 1117.6573x; 1117.6573x over previous
import functools

import jax
import jax.numpy as jnp
from jax import lax
from jax.experimental import pallas as pl
from jax.experimental.pallas import tpu as pltpu
from jax.experimental.pallas import tpu_sc as plsc

N = 2097152
NROWS = 16384
NRAYS = 65536
_LANE_STEPS = (1, 2, 4, 8, 16, 32, 64)
_INF = float("inf")


def _tri_ge(dtype=jnp.float32):
    r = lax.broadcasted_iota(jnp.int32, (128, 128), 0)
    c = lax.broadcasted_iota(jnp.int32, (128, 128), 1)
    return (r <= c).astype(dtype)


def _tri_strict(dtype=jnp.float32):
    r = lax.broadcasted_iota(jnp.int32, (128, 128), 0)
    c = lax.broadcasted_iota(jnp.int32, (128, 128), 1)
    return (c < r).astype(dtype)


def _dot(a, b):
    return jax.lax.dot_general(
        a, b, (((1,), (0,)), ((), ())), preferred_element_type=jnp.float32,
        precision=lax.Precision.HIGHEST)


def _log1m(a):
    return jnp.log(jnp.clip(1.0 - a, 1e-6, 1.0))


def _roll_fill_lanes(x, k, fill):
    r = pltpu.roll(x, k, axis=1)
    lane = lax.broadcasted_iota(jnp.int32, x.shape, 1)
    return jnp.where(lane >= k, r, fill)


def _roll_fill_sub(x, k, fill):
    r = pltpu.roll(x, k, axis=0)
    row = lax.broadcasted_iota(jnp.int32, x.shape, 0)
    return jnp.where(row >= k, r, fill)


def _p1_body(a_ref, bs_ref):
    l = _log1m(a_ref[...])
    rw1 = _dot(l, _tri_ge())
    bs_ref[...] = rw1[:, 127:128]


def _pass1(a2d):
    return pl.pallas_call(
        _p1_body,
        grid=(32,),
        in_specs=[pl.BlockSpec((512, 128), lambda b: (b, 0))],
        out_specs=pl.BlockSpec((512, 1), lambda b: (b, 0)),
        out_shape=jax.ShapeDtypeStruct((NROWS, 1), jnp.float32),
        compiler_params=pltpu.CompilerParams(
            dimension_semantics=("parallel",)),
    )(a2d)


def _p2_body(a_ref, m_ref, bs_ref, t_ref, sseg_ref, s1_ref, carry_ref):
    b = pl.program_id(0)

    @pl.when(b == 0)
    def _init():
        bs = bs_ref[...]
        rw2 = _dot(bs, _tri_ge())
        rowtot = rw2[:, 127:128]
        l3exc = _dot(_tri_strict(), rowtot)
        s1inc = rw2 + l3exc
        rolled = pltpu.roll(s1inc, 1, axis=1)
        prevrow = pltpu.roll(s1inc[:, 127:128], 1, axis=0)
        lane0 = lax.broadcasted_iota(jnp.int32, (128, 128), 1) == 0
        row0 = lax.broadcasted_iota(jnp.int32, (128, 1), 0) == 0
        first = jnp.where(row0, 0.0, prevrow)
        s1_ref[...] = jnp.where(lane0, jnp.broadcast_to(first, (128, 128)),
                                rolled)
        carry_ref[...] = jnp.zeros((8, 128), jnp.float32)

    a = a_ref[...]
    l = _log1m(a)
    rw1 = _dot(l, _tri_ge())

    onehot = (lax.broadcasted_iota(jnp.int32, (1, 128), 1) == b)
    rowvec = _dot(onehot.astype(jnp.float32), s1_ref[...])
    eye = (lax.broadcasted_iota(jnp.int32, (128, 128), 0)
           == lax.broadcasted_iota(jnp.int32, (128, 128), 1)).astype(jnp.float32)
    offs = jax.lax.dot_general(eye, rowvec, (((1,), (1,)), ((), ())),
                               preferred_element_type=jnp.float32,
                               precision=lax.Precision.HIGHEST)

    E = (rw1 + offs) - l

    mask = m_ref[...] > 0.5
    okf = m_ref[...]
    vE = jnp.where(mask, E, 0.0)
    for k in _LANE_STEPS:
        rv = pltpu.roll(vE, k, axis=1)
        rok = _roll_fill_lanes(okf, k, 0.0)
        vE = jnp.where(okf > 0.5, vE, rv)
        okf = jnp.maximum(okf, rok)

    cg = carry_ref[0:1, 0:1]
    cok = carry_ref[1:2, 0:1]
    row0 = lax.broadcasted_iota(jnp.int32, (128, 1), 0) == 0
    sv = pltpu.roll(vE[:, 127:128], 1, axis=0)
    sok = pltpu.roll(okf[:, 127:128], 1, axis=0)
    sv = jnp.where(row0, jnp.broadcast_to(cg, (128, 1)), sv)
    sok = jnp.where(row0, jnp.broadcast_to(cok, (128, 1)), sok)
    for k in _LANE_STEPS:
        rv = pltpu.roll(sv, k, axis=0)
        rok = _roll_fill_sub(sok, k, 0.0)
        sv = jnp.where(sok > 0.5, sv, rv)
        sok = jnp.maximum(sok, rok)
    G = jnp.where(okf > 0.5, vE, jnp.broadcast_to(sv, (128, 128)))

    t_ref[...] = jnp.exp(E - G)

    carry_ref[0:1, 0:1] = jnp.where(okf[127:128, 127:128] > 0.5,
                                    vE[127:128, 127:128], sv[127:128, 0:1])
    carry_ref[1:2, 0:1] = jnp.maximum(cok, jnp.max(okf[:, 127:128],
                                                   axis=0, keepdims=True))

    rowoffs = _dot(_tri_strict(), rw1[:, 127:128])
    Sc = rw1 + rowoffs
    w = jnp.where(mask, Sc - l, _INF)
    for k in _LANE_STEPS:
        w = jnp.minimum(w, _roll_fill_lanes(w, k, _INF))
    cs = carry_ref[2:3, 0:1]
    sw = pltpu.roll(w[:, 127:128], 1, axis=0)
    sw = jnp.where(row0, jnp.broadcast_to(0.0 - cs, (128, 1)), sw)
    for k in _LANE_STEPS:
        sw = jnp.minimum(sw, _roll_fill_sub(sw, k, _INF))
    W = jnp.minimum(w, jnp.broadcast_to(sw, (128, 128)))
    sseg = Sc - W
    sseg_ref[...] = sseg
    carry_ref[2:3, 0:1] = sseg[127:128, 127:128]


def _pass2(a2d, mask2d, bs2d):
    return pl.pallas_call(
        _p2_body,
        grid=(128,),
        in_specs=[
            pl.BlockSpec((128, 128), lambda b: (b, 0)),
            pl.BlockSpec((128, 128), lambda b: (b, 0)),
            pl.BlockSpec((128, 128), lambda b: (0, 0)),
        ],
        out_specs=[
            pl.BlockSpec((128, 128), lambda b: (b, 0)),
            pl.BlockSpec((128, 128), lambda b: (b, 0)),
        ],
        out_shape=[
            jax.ShapeDtypeStruct((NROWS, 128), jnp.float32),
            jax.ShapeDtypeStruct((NROWS, 128), jnp.float32),
        ],
        scratch_shapes=[
            pltpu.VMEM((128, 128), jnp.float32),
            pltpu.VMEM((8, 128), jnp.float32),
        ],
        compiler_params=pltpu.CompilerParams(
            dimension_semantics=("arbitrary",)),
    )(a2d, mask2d, bs2d)


def _sc_mesh():
    return plsc.VectorSubcoreMesh(core_axis_name="c", subcore_axis_name="s")


def _sc_wid():
    info = plsc.get_sparse_core_info()
    return lax.axis_index("s") * info.num_cores + lax.axis_index("c")


def _sc_scatter_mask(cu_lo, mask0):
    @functools.partial(
        pl.kernel,
        out_type=(),
        mesh=_sc_mesh(),
        scratch_types=[
            pltpu.VMEM((16, 128), jnp.int32),
            pltpu.VMEM((128,), jnp.float32),
            pltpu.SemaphoreType.DMA,
        ],
    )
    def body(cu_hbm, mask_hbm, idx_v, ones_v, sem):
        base = _sc_wid() * 2048
        for i in range(8):
            ones_v[pl.ds(i * 16, 16)] = jnp.full((16,), 1.0, jnp.float32)
        for j in range(16):
            pltpu.sync_copy(cu_hbm.at[pl.ds(base + j * 128, 128)],
                            idx_v.at[j])
        for j in range(16):
            pltpu.async_copy(ones_v, mask_hbm.at[idx_v.at[j]], sem).wait()

    mref = jax.new_ref(mask0)
    body(cu_lo, mref)
    return mref[...]


def _sc_gather_bg(cu_lo, cu_hi, sseg_flat):
    @functools.partial(
        pl.kernel,
        out_type=jax.ShapeDtypeStruct((NRAYS,), jnp.float32),
        mesh=_sc_mesh(),
        scratch_types=[
            pltpu.VMEM((2048,), jnp.int32),
            pltpu.VMEM((2048,), jnp.int32),
            pltpu.VMEM((2048,), jnp.int32),
            pltpu.VMEM((2048,), jnp.float32),
            pltpu.VMEM((2048,), jnp.float32),
            pltpu.SemaphoreType.DMA,
        ],
    )
    def body(lo_hbm, hi_hbm, sseg_hbm, out_hbm,
             lo_v, hi_v, idx_v, g_v, out_v, sem):
        base = _sc_wid() * 2048
        pltpu.sync_copy(lo_hbm.at[pl.ds(base, 2048)], lo_v)
        pltpu.sync_copy(hi_hbm.at[pl.ds(base, 2048)], hi_v)
        for i in range(128):
            h = hi_v[pl.ds(i * 16, 16)]
            idx_v[pl.ds(i * 16, 16)] = jnp.maximum(h - 1, 0)
        for j in range(16):
            pltpu.async_copy(sseg_hbm.at[idx_v.at[pl.ds(j * 128, 128)]],
                             g_v.at[pl.ds(j * 128, 128)], sem).wait()
        for i in range(128):
            sl = pl.ds(i * 16, 16)
            empty = hi_v[sl] == lo_v[sl]
            out_v[sl] = jnp.where(empty, jnp.float32(1.0), jnp.exp(g_v[sl]))
        pltpu.sync_copy(out_v, out_hbm.at[pl.ds(base, 2048)])

    return body(cu_lo, cu_hi, sseg_flat)


def kernel(alpha, cu_seqlens):
    a2d = alpha.reshape(NROWS, 128)
    cu_lo = lax.slice(cu_seqlens, (0,), (NRAYS,))
    cu_hi = lax.slice(cu_seqlens, (1,), (NRAYS + 1,))

    bs = _pass1(a2d)
    mask = _sc_scatter_mask(cu_lo, jnp.zeros((N,), jnp.float32))
    t2d, sseg2d = _pass2(a2d, mask.reshape(NROWS, 128), bs.reshape(128, 128))
    bg = _sc_gather_bg(cu_lo, cu_hi, sseg2d.reshape(N))
    return t2d.reshape(N, 1), bg.reshape(NRAYS, 1)


# speedup vs baseline: 1150.7946x; 1.0296x over previous
import functools

import jax
import jax.numpy as jnp
from jax import lax
from jax.experimental import pallas as pl
from jax.experimental.pallas import tpu as pltpu
from jax.experimental.pallas import tpu_sc as plsc

N = 2097152
NROWS = 16384
NRAYS = 65536
_LANE_STEPS = (1, 2, 4, 8, 16, 32, 64)
_INF = float("inf")


def _tri_ge(dtype=jnp.float32):
    r = lax.broadcasted_iota(jnp.int32, (128, 128), 0)
    c = lax.broadcasted_iota(jnp.int32, (128, 128), 1)
    return (r <= c).astype(dtype)


def _tri_strict(dtype=jnp.float32):
    r = lax.broadcasted_iota(jnp.int32, (128, 128), 0)
    c = lax.broadcasted_iota(jnp.int32, (128, 128), 1)
    return (c < r).astype(dtype)


def _dot(a, b):
    return jax.lax.dot_general(
        a, b, (((1,), (0,)), ((), ())), preferred_element_type=jnp.float32,
        precision=lax.Precision.HIGHEST)


def _log1m(a):
    return jnp.log(jnp.clip(1.0 - a, 1e-6, 1.0))


def _roll_fill_lanes(x, k, fill):
    r = pltpu.roll(x, k, axis=1)
    lane = lax.broadcasted_iota(jnp.int32, x.shape, 1)
    return jnp.where(lane >= k, r, fill)


def _roll_fill_sub(x, k, fill):
    r = pltpu.roll(x, k, axis=0)
    row = lax.broadcasted_iota(jnp.int32, x.shape, 0)
    return jnp.where(row >= k, r, fill)


def _p1_body(a_ref, bs_ref):
    l = _log1m(a_ref[...])
    rw1 = _dot(l, _tri_ge())
    bs_ref[...] = rw1[:, 127:128]


def _pass1(a2d):
    return pl.pallas_call(
        _p1_body,
        grid=(32,),
        in_specs=[pl.BlockSpec((512, 128), lambda b: (b, 0))],
        out_specs=pl.BlockSpec((512, 1), lambda b: (b, 0)),
        out_shape=jax.ShapeDtypeStruct((NROWS, 1), jnp.float32),
        compiler_params=pltpu.CompilerParams(
            dimension_semantics=("parallel",)),
    )(a2d)


def _p2_body(a_ref, m_ref, bs_ref, t_ref, sseg_ref, s1_ref, carry_ref):
    b = pl.program_id(0)

    @pl.when(b == 0)
    def _init():
        bs = bs_ref[...]
        rw2 = _dot(bs, _tri_ge())
        rowtot = rw2[:, 127:128]
        l3exc = _dot(_tri_strict(), rowtot)
        s1inc = rw2 + l3exc
        rolled = pltpu.roll(s1inc, 1, axis=1)
        prevrow = pltpu.roll(s1inc[:, 127:128], 1, axis=0)
        lane0 = lax.broadcasted_iota(jnp.int32, (128, 128), 1) == 0
        row0 = lax.broadcasted_iota(jnp.int32, (128, 1), 0) == 0
        first = jnp.where(row0, 0.0, prevrow)
        s1_ref[...] = jnp.where(lane0, jnp.broadcast_to(first, (128, 128)),
                                rolled)
        carry_ref[...] = jnp.zeros((8, 128), jnp.float32)

    a = a_ref[...]
    l = _log1m(a)
    rw1 = _dot(l, _tri_ge())

    onehot = (lax.broadcasted_iota(jnp.int32, (1, 128), 1) == b)
    rowvec = _dot(onehot.astype(jnp.float32), s1_ref[...])
    eye = (lax.broadcasted_iota(jnp.int32, (128, 128), 0)
           == lax.broadcasted_iota(jnp.int32, (128, 128), 1)).astype(jnp.float32)
    offs = jax.lax.dot_general(eye, rowvec, (((1,), (1,)), ((), ())),
                               preferred_element_type=jnp.float32,
                               precision=lax.Precision.HIGHEST)

    E = (rw1 + offs) - l

    mask = m_ref[...] > 0.5
    okf = m_ref[...]
    vE = jnp.where(mask, E, 0.0)
    for k in _LANE_STEPS:
        rv = pltpu.roll(vE, k, axis=1)
        rok = _roll_fill_lanes(okf, k, 0.0)
        vE = jnp.where(okf > 0.5, vE, rv)
        okf = jnp.maximum(okf, rok)

    cg = carry_ref[0:1, 0:1]
    cok = carry_ref[1:2, 0:1]
    row0 = lax.broadcasted_iota(jnp.int32, (128, 1), 0) == 0
    sv = pltpu.roll(vE[:, 127:128], 1, axis=0)
    sok = pltpu.roll(okf[:, 127:128], 1, axis=0)
    sv = jnp.where(row0, jnp.broadcast_to(cg, (128, 1)), sv)
    sok = jnp.where(row0, jnp.broadcast_to(cok, (128, 1)), sok)
    for k in _LANE_STEPS:
        rv = pltpu.roll(sv, k, axis=0)
        rok = _roll_fill_sub(sok, k, 0.0)
        sv = jnp.where(sok > 0.5, sv, rv)
        sok = jnp.maximum(sok, rok)
    G = jnp.where(okf > 0.5, vE, jnp.broadcast_to(sv, (128, 128)))

    t_ref[...] = jnp.exp(E - G)

    carry_ref[0:1, 0:1] = jnp.where(okf[127:128, 127:128] > 0.5,
                                    vE[127:128, 127:128], sv[127:128, 0:1])
    carry_ref[1:2, 0:1] = jnp.maximum(cok, jnp.max(okf[:, 127:128],
                                                   axis=0, keepdims=True))

    rowoffs = _dot(_tri_strict(), rw1[:, 127:128])
    Sc = rw1 + rowoffs
    w = jnp.where(mask, Sc - l, _INF)
    for k in _LANE_STEPS:
        w = jnp.minimum(w, _roll_fill_lanes(w, k, _INF))
    cs = carry_ref[2:3, 0:1]
    sw = pltpu.roll(w[:, 127:128], 1, axis=0)
    sw = jnp.where(row0, jnp.broadcast_to(0.0 - cs, (128, 1)), sw)
    for k in _LANE_STEPS:
        sw = jnp.minimum(sw, _roll_fill_sub(sw, k, _INF))
    W = jnp.minimum(w, jnp.broadcast_to(sw, (128, 128)))
    sseg = Sc - W
    sseg_ref[...] = sseg
    carry_ref[2:3, 0:1] = sseg[127:128, 127:128]


def _pass2(a2d, mask2d, bs2d):
    return pl.pallas_call(
        _p2_body,
        grid=(128,),
        in_specs=[
            pl.BlockSpec((128, 128), lambda b: (b, 0)),
            pl.BlockSpec((128, 128), lambda b: (b, 0)),
            pl.BlockSpec((128, 128), lambda b: (0, 0)),
        ],
        out_specs=[
            pl.BlockSpec((128, 128), lambda b: (b, 0)),
            pl.BlockSpec((128, 128), lambda b: (b, 0)),
        ],
        out_shape=[
            jax.ShapeDtypeStruct((NROWS, 128), jnp.float32),
            jax.ShapeDtypeStruct((NROWS, 128), jnp.float32),
        ],
        scratch_shapes=[
            pltpu.VMEM((128, 128), jnp.float32),
            pltpu.VMEM((8, 128), jnp.float32),
        ],
        compiler_params=pltpu.CompilerParams(
            dimension_semantics=("arbitrary",)),
    )(a2d, mask2d, bs2d)


def _sc_mesh():
    return plsc.VectorSubcoreMesh(core_axis_name="c", subcore_axis_name="s")


def _sc_wid():
    info = plsc.get_sparse_core_info()
    return lax.axis_index("s") * info.num_cores + lax.axis_index("c")


def _sc_scatter_mask(cu_lo, mask0):
    @functools.partial(
        pl.kernel,
        out_type=(),
        mesh=_sc_mesh(),
        scratch_types=[
            pltpu.VMEM((16, 128), jnp.int32),
            pltpu.VMEM((128,), jnp.float32),
            pltpu.SemaphoreType.DMA,
        ],
    )
    def body(cu_hbm, mask_hbm, idx_v, ones_v, sem):
        base = _sc_wid() * 2048
        for i in range(8):
            ones_v[pl.ds(i * 16, 16)] = jnp.full((16,), 1.0, jnp.float32)
        for j in range(16):
            pltpu.sync_copy(cu_hbm.at[pl.ds(base + j * 128, 128)],
                            idx_v.at[j])
        copies = [pltpu.async_copy(ones_v, mask_hbm.at[idx_v.at[j]], sem)
                  for j in range(16)]
        for c in copies:
            c.wait()

    mref = jax.new_ref(mask0)
    body(cu_lo, mref)
    return mref[...]


def _sc_gather_bg(cu_lo, cu_hi, sseg_flat):
    @functools.partial(
        pl.kernel,
        out_type=jax.ShapeDtypeStruct((NRAYS,), jnp.float32),
        mesh=_sc_mesh(),
        scratch_types=[
            pltpu.VMEM((2048,), jnp.int32),
            pltpu.VMEM((2048,), jnp.int32),
            pltpu.VMEM((2048,), jnp.int32),
            pltpu.VMEM((2048,), jnp.float32),
            pltpu.VMEM((2048,), jnp.float32),
            pltpu.SemaphoreType.DMA,
        ],
    )
    def body(lo_hbm, hi_hbm, sseg_hbm, out_hbm,
             lo_v, hi_v, idx_v, g_v, out_v, sem):
        base = _sc_wid() * 2048
        pltpu.sync_copy(lo_hbm.at[pl.ds(base, 2048)], lo_v)
        pltpu.sync_copy(hi_hbm.at[pl.ds(base, 2048)], hi_v)
        for i in range(128):
            h = hi_v[pl.ds(i * 16, 16)]
            idx_v[pl.ds(i * 16, 16)] = jnp.maximum(h - 1, 0)
        copies = [
            pltpu.async_copy(sseg_hbm.at[idx_v.at[pl.ds(j * 128, 128)]],
                             g_v.at[pl.ds(j * 128, 128)], sem)
            for j in range(16)
        ]
        for c in copies:
            c.wait()
        for i in range(128):
            sl = pl.ds(i * 16, 16)
            empty = hi_v[sl] == lo_v[sl]
            out_v[sl] = jnp.where(empty, jnp.float32(1.0), jnp.exp(g_v[sl]))
        pltpu.sync_copy(out_v, out_hbm.at[pl.ds(base, 2048)])

    return body(cu_lo, cu_hi, sseg_flat)


def kernel(alpha, cu_seqlens):
    a2d = alpha.reshape(NROWS, 128)
    cu_lo = lax.slice(cu_seqlens, (0,), (NRAYS,))
    cu_hi = lax.slice(cu_seqlens, (1,), (NRAYS + 1,))

    bs = _pass1(a2d)
    mask = _sc_scatter_mask(cu_lo, jnp.zeros((N,), jnp.float32))
    t2d, sseg2d = _pass2(a2d, mask.reshape(NROWS, 128), bs.reshape(128, 128))
    bg = _sc_gather_bg(cu_lo, cu_hi, sseg2d.reshape(N))
    return t2d.reshape(N, 1), bg.reshape(NRAYS, 1)


# speedup vs baseline: 1178.2388x; 1.0238x over previous
import functools

import jax
import jax.numpy as jnp
from jax import lax
from jax.experimental import pallas as pl
from jax.experimental.pallas import tpu as pltpu
from jax.experimental.pallas import tpu_sc as plsc

N = 2097152
NROWS = 16384
NRAYS = 65536
_LANE_STEPS = (1, 2, 4, 8, 16, 32, 64)
_INF = float("inf")


def _tri_ge(dtype=jnp.float32):
    r = lax.broadcasted_iota(jnp.int32, (128, 128), 0)
    c = lax.broadcasted_iota(jnp.int32, (128, 128), 1)
    return (r <= c).astype(dtype)


def _tri_strict(dtype=jnp.float32):
    r = lax.broadcasted_iota(jnp.int32, (128, 128), 0)
    c = lax.broadcasted_iota(jnp.int32, (128, 128), 1)
    return (c < r).astype(dtype)


def _dot(a, b):
    return jax.lax.dot_general(
        a, b, (((1,), (0,)), ((), ())), preferred_element_type=jnp.float32,
        precision=lax.Precision.HIGHEST)


def _log1m(a):
    return jnp.log(jnp.clip(1.0 - a, 1e-6, 1.0))


def _roll_fill_lanes(x, k, fill):
    r = pltpu.roll(x, k, axis=1)
    lane = lax.broadcasted_iota(jnp.int32, x.shape, 1)
    return jnp.where(lane >= k, r, fill)


def _roll_fill_sub(x, k, fill):
    r = pltpu.roll(x, k, axis=0)
    row = lax.broadcasted_iota(jnp.int32, x.shape, 0)
    return jnp.where(row >= k, r, fill)


def _p1_body(a_ref, bs_ref):
    l = _log1m(a_ref[...])
    rw1 = _dot(l, _tri_ge())
    bs_ref[...] = rw1[:, 127:128]


def _pass1(a2d):
    return pl.pallas_call(
        _p1_body,
        grid=(32,),
        in_specs=[pl.BlockSpec((512, 128), lambda b: (b, 0))],
        out_specs=pl.BlockSpec((512, 1), lambda b: (b, 0)),
        out_shape=jax.ShapeDtypeStruct((NROWS, 1), jnp.float32),
        compiler_params=pltpu.CompilerParams(
            dimension_semantics=("parallel",)),
    )(a2d)


def _p2_body(a_ref, m_ref, bs_ref, t_ref, sseg_ref, s1_ref, carry_ref):
    b = pl.program_id(0)

    @pl.when(b == 0)
    def _init():
        bs = bs_ref[...]
        rw2 = _dot(bs, _tri_ge())
        rowtot = rw2[:, 127:128]
        l3exc = _dot(_tri_strict(), rowtot)
        s1inc = rw2 + l3exc
        rolled = pltpu.roll(s1inc, 1, axis=1)
        prevrow = pltpu.roll(s1inc[:, 127:128], 1, axis=0)
        lane0 = lax.broadcasted_iota(jnp.int32, (128, 128), 1) == 0
        row0 = lax.broadcasted_iota(jnp.int32, (128, 1), 0) == 0
        first = jnp.where(row0, 0.0, prevrow)
        s1_ref[...] = jnp.where(lane0, jnp.broadcast_to(first, (128, 128)),
                                rolled)
        carry_ref[...] = jnp.zeros((8, 128), jnp.float32)

    a = a_ref[...]
    l = _log1m(a)
    rw1 = _dot(l, _tri_ge())

    onehot = (lax.broadcasted_iota(jnp.int32, (1, 128), 1) == b)
    rowvec = _dot(onehot.astype(jnp.float32), s1_ref[...])
    eye = (lax.broadcasted_iota(jnp.int32, (128, 128), 0)
           == lax.broadcasted_iota(jnp.int32, (128, 128), 1)).astype(jnp.float32)
    offs = jax.lax.dot_general(eye, rowvec, (((1,), (1,)), ((), ())),
                               preferred_element_type=jnp.float32,
                               precision=lax.Precision.HIGHEST)

    E = (rw1 + offs) - l

    mask = m_ref[...] > 0.5
    okf = m_ref[...]
    vE = jnp.where(mask, E, 0.0)
    for k in _LANE_STEPS:
        rv = pltpu.roll(vE, k, axis=1)
        rok = _roll_fill_lanes(okf, k, 0.0)
        vE = jnp.where(okf > 0.5, vE, rv)
        okf = jnp.maximum(okf, rok)

    cg = carry_ref[0:1, 0:1]
    cok = carry_ref[1:2, 0:1]
    row0 = lax.broadcasted_iota(jnp.int32, (128, 1), 0) == 0
    sv = pltpu.roll(vE[:, 127:128], 1, axis=0)
    sok = pltpu.roll(okf[:, 127:128], 1, axis=0)
    sv = jnp.where(row0, jnp.broadcast_to(cg, (128, 1)), sv)
    sok = jnp.where(row0, jnp.broadcast_to(cok, (128, 1)), sok)
    for k in _LANE_STEPS:
        rv = pltpu.roll(sv, k, axis=0)
        rok = _roll_fill_sub(sok, k, 0.0)
        sv = jnp.where(sok > 0.5, sv, rv)
        sok = jnp.maximum(sok, rok)
    G = jnp.where(okf > 0.5, vE, jnp.broadcast_to(sv, (128, 128)))

    t_ref[...] = jnp.exp(E - G)

    carry_ref[0:1, 0:1] = jnp.where(okf[127:128, 127:128] > 0.5,
                                    vE[127:128, 127:128], sv[127:128, 0:1])
    carry_ref[1:2, 0:1] = jnp.maximum(cok, jnp.max(okf[:, 127:128],
                                                   axis=0, keepdims=True))

    rowoffs = _dot(_tri_strict(), rw1[:, 127:128])
    Sc = rw1 + rowoffs
    w = jnp.where(mask, Sc - l, _INF)
    for k in _LANE_STEPS:
        w = jnp.minimum(w, _roll_fill_lanes(w, k, _INF))
    cs = carry_ref[2:3, 0:1]
    sw = pltpu.roll(w[:, 127:128], 1, axis=0)
    sw = jnp.where(row0, jnp.broadcast_to(0.0 - cs, (128, 1)), sw)
    for k in _LANE_STEPS:
        sw = jnp.minimum(sw, _roll_fill_sub(sw, k, _INF))
    W = jnp.minimum(w, jnp.broadcast_to(sw, (128, 128)))
    sseg = Sc - W
    sseg_ref[...] = sseg
    carry_ref[2:3, 0:1] = sseg[127:128, 127:128]


def _pass2(a2d, mask2d, bs2d):
    return pl.pallas_call(
        _p2_body,
        grid=(128,),
        in_specs=[
            pl.BlockSpec((128, 128), lambda b: (b, 0)),
            pl.BlockSpec((128, 128), lambda b: (b, 0)),
            pl.BlockSpec((128, 128), lambda b: (0, 0)),
        ],
        out_specs=[
            pl.BlockSpec((128, 128), lambda b: (b, 0)),
            pl.BlockSpec((128, 128), lambda b: (b, 0)),
        ],
        out_shape=[
            jax.ShapeDtypeStruct((NROWS, 128), jnp.float32),
            jax.ShapeDtypeStruct((NROWS, 128), jnp.float32),
        ],
        scratch_shapes=[
            pltpu.VMEM((128, 128), jnp.float32),
            pltpu.VMEM((8, 128), jnp.float32),
        ],
        compiler_params=pltpu.CompilerParams(
            dimension_semantics=("arbitrary",)),
    )(a2d, mask2d, bs2d)


def _sc_mesh():
    return plsc.VectorSubcoreMesh(core_axis_name="c", subcore_axis_name="s")


def _sc_wid():
    info = plsc.get_sparse_core_info()
    return lax.axis_index("s") * info.num_cores + lax.axis_index("c")


def _sc_scatter_mask(cu_lo, mask0):
    @functools.partial(
        pl.kernel,
        out_type=(),
        mesh=_sc_mesh(),
        scratch_types=[
            pltpu.VMEM((2048,), jnp.int32),
            pltpu.VMEM((16, 128), jnp.int32),
            pltpu.VMEM((128,), jnp.float32),
            pltpu.SemaphoreType.DMA,
        ],
    )
    def body(cu_hbm, mask_hbm, cu_v, idx_v, ones_v, sem):
        base = _sc_wid() * 2048
        for i in range(8):
            ones_v[pl.ds(i * 16, 16)] = jnp.full((16,), 1.0, jnp.float32)
        pltpu.sync_copy(cu_hbm.at[pl.ds(base, 2048)], cu_v)
        for j in range(16):
            for k in range(8):
                idx_v[j, pl.ds(k * 16, 16)] = cu_v[pl.ds(j * 128 + k * 16, 16)]
        copies = [pltpu.async_copy(ones_v, mask_hbm.at[idx_v.at[j]], sem)
                  for j in range(16)]
        for c in copies:
            c.wait()

    mref = jax.new_ref(mask0)
    body(cu_lo, mref)
    return mref[...]


def _sc_gather_bg(cu_lo, cu_hi, sseg_flat):
    @functools.partial(
        pl.kernel,
        out_type=jax.ShapeDtypeStruct((NRAYS,), jnp.float32),
        mesh=_sc_mesh(),
        scratch_types=[
            pltpu.VMEM((2048,), jnp.int32),
            pltpu.VMEM((2048,), jnp.int32),
            pltpu.VMEM((2048,), jnp.int32),
            pltpu.VMEM((2048,), jnp.float32),
            pltpu.VMEM((2048,), jnp.float32),
            pltpu.SemaphoreType.DMA,
        ],
    )
    def body(lo_hbm, hi_hbm, sseg_hbm, out_hbm,
             lo_v, hi_v, idx_v, g_v, out_v, sem):
        base = _sc_wid() * 2048
        pltpu.sync_copy(lo_hbm.at[pl.ds(base, 2048)], lo_v)
        pltpu.sync_copy(hi_hbm.at[pl.ds(base, 2048)], hi_v)
        for i in range(128):
            h = hi_v[pl.ds(i * 16, 16)]
            idx_v[pl.ds(i * 16, 16)] = jnp.maximum(h - 1, 0)
        copies = [
            pltpu.async_copy(sseg_hbm.at[idx_v.at[pl.ds(j * 128, 128)]],
                             g_v.at[pl.ds(j * 128, 128)], sem)
            for j in range(16)
        ]
        for c in copies:
            c.wait()
        for i in range(128):
            sl = pl.ds(i * 16, 16)
            empty = hi_v[sl] == lo_v[sl]
            out_v[sl] = jnp.where(empty, jnp.float32(1.0), jnp.exp(g_v[sl]))
        pltpu.sync_copy(out_v, out_hbm.at[pl.ds(base, 2048)])

    return body(cu_lo, cu_hi, sseg_flat)


def kernel(alpha, cu_seqlens):
    a2d = alpha.reshape(NROWS, 128)
    cu_lo = lax.slice(cu_seqlens, (0,), (NRAYS,))
    cu_hi = lax.slice(cu_seqlens, (1,), (NRAYS + 1,))

    bs = _pass1(a2d)
    mask = _sc_scatter_mask(cu_lo, jnp.zeros((N,), jnp.float32))
    t2d, sseg2d = _pass2(a2d, mask.reshape(NROWS, 128), bs.reshape(128, 128))
    bg = _sc_gather_bg(cu_lo, cu_hi, sseg2d.reshape(N))
    return t2d.reshape(N, 1), bg.reshape(NRAYS, 1)


# speedup vs baseline: 1724.1623x; 1.4633x over previous
import functools

import jax
import jax.numpy as jnp
from jax import lax
from jax.experimental import pallas as pl
from jax.experimental.pallas import tpu as pltpu
from jax.experimental.pallas import tpu_sc as plsc

N = 2097152
NROWS = 16384
NRAYS = 65536
_LANE_STEPS = (1, 2, 4, 8, 16, 32, 64)
_INF = float("inf")


def _tri_ge(dtype=jnp.float32):
    r = lax.broadcasted_iota(jnp.int32, (128, 128), 0)
    c = lax.broadcasted_iota(jnp.int32, (128, 128), 1)
    return (r <= c).astype(dtype)


def _tri_strict(dtype=jnp.float32):
    r = lax.broadcasted_iota(jnp.int32, (128, 128), 0)
    c = lax.broadcasted_iota(jnp.int32, (128, 128), 1)
    return (c < r).astype(dtype)


def _dot(a, b):
    return jax.lax.dot_general(
        a, b, (((1,), (0,)), ((), ())), preferred_element_type=jnp.float32,
        precision=lax.Precision.HIGHEST)


def _log1m(a):
    return jnp.log(jnp.clip(1.0 - a, 1e-6, 1.0))


def _roll_fill_lanes(x, k, fill):
    r = pltpu.roll(x, k, axis=1)
    lane = lax.broadcasted_iota(jnp.int32, x.shape, 1)
    return jnp.where(lane >= k, r, fill)


def _roll_fill_sub(x, k, fill):
    r = pltpu.roll(x, k, axis=0)
    row = lax.broadcasted_iota(jnp.int32, x.shape, 0)
    return jnp.where(row >= k, r, fill)


def _p1_body(a_ref, bs_ref, rw1_ref, l_ref):
    l = _log1m(a_ref[...])
    rw1 = _dot(l, _tri_ge())
    bs_ref[...] = rw1[:, 127:128]
    rw1_ref[...] = rw1
    l_ref[...] = l


def _pass1(a2d):
    return pl.pallas_call(
        _p1_body,
        grid=(32,),
        in_specs=[pl.BlockSpec((512, 128), lambda b: (b, 0))],
        out_specs=[
            pl.BlockSpec((512, 1), lambda b: (b, 0)),
            pl.BlockSpec((512, 128), lambda b: (b, 0)),
            pl.BlockSpec((512, 128), lambda b: (b, 0)),
        ],
        out_shape=[
            jax.ShapeDtypeStruct((NROWS, 1), jnp.float32),
            jax.ShapeDtypeStruct((NROWS, 128), jnp.float32),
            jax.ShapeDtypeStruct((NROWS, 128), jnp.float32),
        ],
        compiler_params=pltpu.CompilerParams(
            dimension_semantics=("parallel",)),
    )(a2d)


def _p2_body(rw1_ref, l_ref, m_ref, bs_ref, t_ref, sseg_ref, s1_ref,
             carry_ref):
    b = pl.program_id(0)

    @pl.when(b == 0)
    def _init():
        bs = bs_ref[...]
        rw2 = _dot(bs, _tri_ge())
        rowtot = rw2[:, 127:128]
        l3exc = _dot(_tri_strict(), rowtot)
        s1inc = rw2 + l3exc
        rolled = pltpu.roll(s1inc, 1, axis=1)
        prevrow = pltpu.roll(s1inc[:, 127:128], 1, axis=0)
        lane0 = lax.broadcasted_iota(jnp.int32, (128, 128), 1) == 0
        row0 = lax.broadcasted_iota(jnp.int32, (128, 1), 0) == 0
        first = jnp.where(row0, 0.0, prevrow)
        s1_ref[...] = jnp.where(lane0, jnp.broadcast_to(first, (128, 128)),
                                rolled)
        carry_ref[...] = jnp.zeros((8, 128), jnp.float32)

    l = l_ref[...]
    rw1 = rw1_ref[...]

    onehot = (lax.broadcasted_iota(jnp.int32, (1, 128), 1) == b)
    rowvec = _dot(onehot.astype(jnp.float32), s1_ref[...])
    eye = (lax.broadcasted_iota(jnp.int32, (128, 128), 0)
           == lax.broadcasted_iota(jnp.int32, (128, 128), 1)).astype(jnp.float32)
    offs = jax.lax.dot_general(eye, rowvec, (((1,), (1,)), ((), ())),
                               preferred_element_type=jnp.float32,
                               precision=lax.Precision.HIGHEST)

    E = (rw1 + offs) - l

    wds = m_ref[...]
    lane = lax.broadcasted_iota(jnp.int32, (128, 128), 1)
    w01 = jnp.where(lane < 32,
                    jnp.broadcast_to(wds[:, 0:1], (128, 128)),
                    jnp.broadcast_to(wds[:, 1:2], (128, 128)))
    w23 = jnp.where(lane < 96,
                    jnp.broadcast_to(wds[:, 2:3], (128, 128)),
                    jnp.broadcast_to(wds[:, 3:4], (128, 128)))
    wsel = jnp.where(lane < 64, w01, w23)
    mask = (lax.shift_right_logical(wsel, lane & 31) & 1) == 1

    vE = jnp.where(mask, E, _INF)
    for k in _LANE_STEPS:
        vE = jnp.minimum(vE, _roll_fill_lanes(vE, k, _INF))

    cg = carry_ref[0:1, 0:1]
    row0 = lax.broadcasted_iota(jnp.int32, (128, 1), 0) == 0
    sv = pltpu.roll(vE[:, 127:128], 1, axis=0)
    sv = jnp.where(row0, jnp.broadcast_to(cg, (128, 1)), sv)
    for k in _LANE_STEPS:
        sv = jnp.minimum(sv, _roll_fill_sub(sv, k, _INF))
    G = jnp.minimum(vE, jnp.broadcast_to(sv, (128, 128)))

    t_ref[...] = jnp.exp(E - G)

    carry_ref[0:1, 0:1] = jnp.minimum(sv[127:128, 0:1], vE[127:128, 127:128])

    rowoffs = _dot(_tri_strict(), rw1[:, 127:128])
    Sc = rw1 + rowoffs
    w = jnp.where(mask, Sc - l, _INF)
    for k in _LANE_STEPS:
        w = jnp.minimum(w, _roll_fill_lanes(w, k, _INF))
    cs = carry_ref[2:3, 0:1]
    sw = pltpu.roll(w[:, 127:128], 1, axis=0)
    sw = jnp.where(row0, jnp.broadcast_to(0.0 - cs, (128, 1)), sw)
    for k in _LANE_STEPS:
        sw = jnp.minimum(sw, _roll_fill_sub(sw, k, _INF))
    W = jnp.minimum(w, jnp.broadcast_to(sw, (128, 128)))
    sseg = Sc - W
    sseg_ref[...] = sseg
    carry_ref[2:3, 0:1] = sseg[127:128, 127:128]


def _pass2(rw1, l2d, mask2d, bs2d):
    return pl.pallas_call(
        _p2_body,
        grid=(128,),
        in_specs=[
            pl.BlockSpec((128, 128), lambda b: (b, 0)),
            pl.BlockSpec((128, 128), lambda b: (b, 0)),
            pl.BlockSpec((128, 4), lambda b: (b, 0)),
            pl.BlockSpec((128, 128), lambda b: (0, 0)),
        ],
        out_specs=[
            pl.BlockSpec((128, 128), lambda b: (b, 0)),
            pl.BlockSpec((128, 128), lambda b: (b, 0)),
        ],
        out_shape=[
            jax.ShapeDtypeStruct((NROWS, 128), jnp.float32),
            jax.ShapeDtypeStruct((NROWS, 128), jnp.float32),
        ],
        scratch_shapes=[
            pltpu.VMEM((128, 128), jnp.float32),
            pltpu.VMEM((8, 128), jnp.float32),
        ],
        compiler_params=pltpu.CompilerParams(
            dimension_semantics=("arbitrary",)),
    )(rw1, l2d, mask2d, bs2d)


def _sc_mesh():
    return plsc.VectorSubcoreMesh(core_axis_name="c", subcore_axis_name="s")


def _sc_wid():
    info = plsc.get_sparse_core_info()
    return lax.axis_index("s") * info.num_cores + lax.axis_index("c")


def _take16(v, idx):
    return lax.gather(
        v, idx[:, None],
        lax.GatherDimensionNumbers(offset_dims=(), collapsed_slice_dims=(0,),
                                   start_index_map=(0,)),
        (1,), mode=lax.GatherScatterMode.PROMISE_IN_BOUNDS)


def _sc_scatter_mask(cu_lo):
    @functools.partial(
        pl.kernel,
        out_type=jax.ShapeDtypeStruct((N // 32,), jnp.int32),
        mesh=_sc_mesh(),
        scratch_types=[
            pltpu.VMEM((4096,), jnp.int32),
            pltpu.VMEM((16,), jnp.int32),
            pltpu.VMEM((32, 128), jnp.int32),
            pltpu.VMEM((32, 128), jnp.int32),
            pltpu.VMEM((2048,), jnp.int32),
            pltpu.VMEM_SHARED((N // 32,), jnp.int32),
            pltpu.SemaphoreType.DMA,
        ],
    )
    def body(cu_hbm, out_hbm, cu_v, bnd_v, widx_v, val_v, zero_v, shared,
             sem):
        cid = lax.axis_index("c")
        sid = lax.axis_index("s")
        base = sid * 4096
        for i in range(128):
            zero_v[pl.ds(i * 16, 16)] = jnp.zeros((16,), jnp.int32)
        pltpu.sync_copy(zero_v, shared.at[pl.ds(sid * 4096, 2048)])
        pltpu.sync_copy(zero_v, shared.at[pl.ds(sid * 4096 + 2048, 2048)])
        pltpu.sync_copy(cu_hbm.at[pl.ds(base, 4096)], cu_v)
        boff = pl.multiple_of(jnp.maximum(base - 16, 0), 16)
        pltpu.sync_copy(cu_hbm.at[pl.ds(boff, 16)], bnd_v)
        lanes = lax.broadcasted_iota(jnp.int32, (16,), 0)
        shift_idx = jnp.maximum(lanes - 1, 0)
        full15 = jnp.full((16,), 15, jnp.int32)
        ones16 = jnp.full((16,), 1, jnp.int32)
        carry = jnp.where(sid == 0, jnp.full((16,), -1, jnp.int32),
                          _take16(bnd_v[...], full15))
        for i in range(256):
            v = cu_v[pl.ds(i * 16, 16)]
            pv = jnp.where(lanes == 0, carry, _take16(v, shift_idx))
            carry = _take16(v, full15)
            dup = v == pv
            word = lax.shift_right_logical(v, 5)
            bitval = lax.shift_left(ones16, v & 31)
            widx_v[i // 8, pl.ds((i % 8) * 16, 16)] = word
            val_v[i // 8, pl.ds((i % 8) * 16, 16)] = jnp.where(
                dup, jnp.zeros((16,), jnp.int32), bitval)
        plsc.subcore_barrier()
        copies = [pltpu.async_copy(val_v.at[j], shared.at[widx_v.at[j]],
                                   sem, add=True) for j in range(32)]
        for c in copies:
            c.wait()
        plsc.subcore_barrier()
        off = cid * 32768 + sid * 2048
        pltpu.async_copy(shared.at[pl.ds(off, 2048)],
                         out_hbm.at[pl.ds(off, 2048)], sem).wait()

    return body(cu_lo)


def _sc_gather_bg(cu_lo, cu_hi, sseg_flat):
    @functools.partial(
        pl.kernel,
        out_type=jax.ShapeDtypeStruct((NRAYS,), jnp.float32),
        mesh=_sc_mesh(),
        scratch_types=[
            pltpu.VMEM((2048,), jnp.int32),
            pltpu.VMEM((2048,), jnp.int32),
            pltpu.VMEM((2048,), jnp.int32),
            pltpu.VMEM((2048,), jnp.float32),
            pltpu.VMEM((2048,), jnp.float32),
            pltpu.SemaphoreType.DMA,
        ],
    )
    def body(lo_hbm, hi_hbm, sseg_hbm, out_hbm,
             lo_v, hi_v, idx_v, g_v, out_v, sem):
        base = _sc_wid() * 2048
        pltpu.sync_copy(lo_hbm.at[pl.ds(base, 2048)], lo_v)
        pltpu.sync_copy(hi_hbm.at[pl.ds(base, 2048)], hi_v)
        for i in range(128):
            h = hi_v[pl.ds(i * 16, 16)]
            idx_v[pl.ds(i * 16, 16)] = jnp.maximum(h - 1, 0)
        copies = [
            pltpu.async_copy(sseg_hbm.at[idx_v.at[pl.ds(j * 128, 128)]],
                             g_v.at[pl.ds(j * 128, 128)], sem)
            for j in range(16)
        ]
        for c in copies:
            c.wait()
        for i in range(128):
            sl = pl.ds(i * 16, 16)
            empty = hi_v[sl] == lo_v[sl]
            out_v[sl] = jnp.where(empty, jnp.float32(1.0), jnp.exp(g_v[sl]))
        pltpu.sync_copy(out_v, out_hbm.at[pl.ds(base, 2048)])

    return body(cu_lo, cu_hi, sseg_flat)


def kernel(alpha, cu_seqlens):
    a2d = alpha.reshape(NROWS, 128)
    cu_lo = lax.slice(cu_seqlens, (0,), (NRAYS,))
    cu_hi = lax.slice(cu_seqlens, (1,), (NRAYS + 1,))

    bs, rw1, l2d = _pass1(a2d)
    words = _sc_scatter_mask(cu_lo)
    t2d, sseg2d = _pass2(rw1, l2d, words.reshape(NROWS, 4),
                         bs.reshape(128, 128))
    bg = _sc_gather_bg(cu_lo, cu_hi, sseg2d.reshape(N))
    return t2d.reshape(N, 1), bg.reshape(NRAYS, 1)
